# baseline (speedup 1.0000x reference)
import jax
import jax.numpy as jnp
from jax.experimental import pallas as pl
from jax.experimental.pallas import tpu as pltpu

NUM_Q = 100000
D = 256


def _copy_body(e_in, p_in, e_out, p_out, sem_e, sem_p):
    ce = pltpu.make_async_copy(e_in, e_out, sem_e)
    cp = pltpu.make_async_copy(p_in, p_out, sem_p)
    ce.start()
    cp.start()
    ce.wait()
    cp.wait()


def kernel(batch_size, query_embed_weight, query_pos_weight):
    out = jax.ShapeDtypeStruct((NUM_Q, D), jnp.float32)
    query_embed, query_pos = pl.pallas_call(
        _copy_body,
        in_specs=[
            pl.BlockSpec(memory_space=pl.ANY),
            pl.BlockSpec(memory_space=pl.ANY),
        ],
        out_specs=[
            pl.BlockSpec(memory_space=pl.ANY),
            pl.BlockSpec(memory_space=pl.ANY),
        ],
        out_shape=[out, out],
        scratch_shapes=[pltpu.SemaphoreType.DMA, pltpu.SemaphoreType.DMA],
    )(query_embed_weight, query_pos_weight)
    return (query_embed, query_pos)


# speedup vs baseline: 1.0008x; 1.0008x over previous
import jax
import jax.numpy as jnp
from jax.experimental import pallas as pl
from jax.experimental.pallas import tpu as pltpu

NUM_Q = 100000
D = 256


NCHUNK = 20
ROWS = NUM_Q // NCHUNK


def _copy_body(e_in, p_in, e_out, p_out, sems):
    copies = []
    for t, (src, dst) in enumerate(((e_in, e_out), (p_in, p_out))):
        for i in range(NCHUNK):
            sl = pl.ds(i * ROWS, ROWS)
            c = pltpu.make_async_copy(src.at[sl], dst.at[sl], sems.at[t, i])
            c.start()
            copies.append(c)
    for c in copies:
        c.wait()


def kernel(batch_size, query_embed_weight, query_pos_weight):
    out = jax.ShapeDtypeStruct((NUM_Q, D), jnp.float32)
    query_embed, query_pos = pl.pallas_call(
        _copy_body,
        in_specs=[
            pl.BlockSpec(memory_space=pl.ANY),
            pl.BlockSpec(memory_space=pl.ANY),
        ],
        out_specs=[
            pl.BlockSpec(memory_space=pl.ANY),
            pl.BlockSpec(memory_space=pl.ANY),
        ],
        out_shape=[out, out],
        scratch_shapes=[pltpu.SemaphoreType.DMA((2, NCHUNK))],
    )(query_embed_weight, query_pos_weight)
    return (query_embed, query_pos)


# speedup vs baseline: 48.1555x; 48.1169x over previous
import jax
import jax.numpy as jnp
from jax.experimental import pallas as pl
from jax.experimental.pallas import tpu as pltpu

NUM_Q = 100000
D = 256
BLOCK = 2000
GRID = NUM_Q // BLOCK


def _copy_body(e_in, p_in, e_out, p_out):
    e_out[...] = e_in[...]
    p_out[...] = p_in[...]


def kernel(batch_size, query_embed_weight, query_pos_weight):
    out = jax.ShapeDtypeStruct((NUM_Q, D), jnp.float32)
    spec = pl.BlockSpec((BLOCK, D), lambda i: (i, 0))
    query_embed, query_pos = pl.pallas_call(
        _copy_body,
        grid=(GRID,),
        in_specs=[spec, spec],
        out_specs=[spec, spec],
        out_shape=[out, out],
    )(query_embed_weight, query_pos_weight)
    return (query_embed, query_pos)


# speedup vs baseline: 48.8775x; 1.0150x over previous
import jax
import jax.numpy as jnp
from jax.experimental import pallas as pl
from jax.experimental.pallas import tpu as pltpu

NUM_Q = 100000
D = 256
BLOCK = 4000
GRID = NUM_Q // BLOCK


def _copy_body(e_in, p_in, e_out, p_out):
    e_out[...] = e_in[...]
    p_out[...] = p_in[...]


def kernel(batch_size, query_embed_weight, query_pos_weight):
    out = jax.ShapeDtypeStruct((NUM_Q, D), jnp.float32)
    spec = pl.BlockSpec((BLOCK, D), lambda i: (i, 0))
    query_embed, query_pos = pl.pallas_call(
        _copy_body,
        grid=(GRID,),
        in_specs=[spec, spec],
        out_specs=[spec, spec],
        out_shape=[out, out],
    )(query_embed_weight, query_pos_weight)
    return (query_embed, query_pos)
